# baseline (speedup 1.0000x reference)
import functools

import jax
import jax.numpy as jnp
from jax import lax
from jax.experimental import pallas as pl
from jax.experimental.pallas import tpu as pltpu
from jax.experimental.pallas import tpu_sc as plsc

_REL_THR = 1000
_NCA = 9
_NCA_PAD = 16
_B = 4
_C = 128
_S = 21760
_N = _S * _NCA_PAD
_N2 = _N // 2
_CHUNK = 17408
_CAND = 1024
_NUM_ANCHORS = _S * _NCA


def _proj_body(x_ref, wpre_ref, bpre_ref, wproj_ref, bproj_ref, lg_ref, ft_ref):
    x = x_ref[0]
    pre = lax.dot_general(x, wpre_ref[...], (((0,), (1,)), ((), ())),
                          preferred_element_type=jnp.float32)
    pre = jnp.maximum(pre + bpre_ref[...][None, :], 0.0)
    lg = lax.dot_general(pre, wproj_ref[...], (((1,), (1,)), ((), ())),
                         preferred_element_type=jnp.float32)
    lg_ref[0] = lg + bproj_ref[...][None, :]
    ft_ref[0] = x.T


def _proj_map(x, wpre, bpre, wprojp, bprojp, sblk):
    B, C, S = x.shape
    nblk = S // sblk
    grid = (B, nblk)
    return pl.pallas_call(
        _proj_body,
        grid=grid,
        in_specs=[
            pl.BlockSpec((1, C, sblk), lambda b, i: (b, 0, i)),
            pl.BlockSpec((C, C), lambda b, i: (0, 0)),
            pl.BlockSpec((C,), lambda b, i: (0,)),
            pl.BlockSpec((_NCA_PAD, C), lambda b, i: (0, 0)),
            pl.BlockSpec((_NCA_PAD,), lambda b, i: (0,)),
        ],
        out_specs=[
            pl.BlockSpec((1, sblk, _NCA_PAD), lambda b, i: (b, i, 0)),
            pl.BlockSpec((1, sblk, C), lambda b, i: (b, i, 0)),
        ],
        out_shape=[
            jax.ShapeDtypeStruct((B, S, _NCA_PAD), jnp.float32),
            jax.ShapeDtypeStruct((B, S, C), jnp.float32),
        ],
    )(x, wpre, bpre, wprojp, bprojp)


def _thresh_body(keys_ref, thr_ref):
    keys = keys_ref[...]

    def step(_, carry):
        lo, hi = carry
        mid = (lo + hi) >> 1
        midf = lax.bitcast_convert_type(mid, jnp.float32)
        cnt = jnp.sum((keys >= midf).astype(jnp.float32), axis=1, keepdims=True)
        ge_k = cnt >= float(_REL_THR)
        return jnp.where(ge_k, mid, lo), jnp.where(ge_k, hi, mid)

    lo0 = jnp.zeros((_B, 1), jnp.int32)
    hi0 = jnp.full((_B, 1), 0x3F800001, jnp.int32)
    lo, _ = lax.fori_loop(0, 31, step, (lo0, hi0))
    thr_ref[...] = jnp.broadcast_to(
        lax.bitcast_convert_type(lo, jnp.float32), (_B, 16))


def _thresh(keys):
    return pl.pallas_call(
        _thresh_body,
        in_specs=[pl.BlockSpec((_B, _N), lambda: (0, 0))],
        out_specs=pl.BlockSpec((_B, 16), lambda: (0, 0)),
        out_shape=jax.ShapeDtypeStruct((_B, 16), jnp.float32),
    )(keys)


def _compact_body(keys_hbm, thr_hbm, ck_hbm, cp_hbm, buf, tv, okeys, opids):
    w = lax.axis_index("s") * 2 + lax.axis_index("c")

    @pl.when(w < 8)
    def _():
        r = w >> 1
        h = w & 1
        pltpu.sync_copy(thr_hbm.at[pl.ds(r * 16, 16)], tv)
        tvec = tv[...]

        zeros16f = jnp.zeros((16,), jnp.float32)
        zeros16i = jnp.zeros((16,), jnp.int32)

        def zinit(i, _):
            okeys[pl.ds(i * 16, 16)] = zeros16f
            opids[pl.ds(i * 16, 16)] = zeros16i
            return 0

        lax.fori_loop(0, _CAND // 16, zinit, 0)

        row_base = r * _N + h * _N2
        pid_base = h * _N2
        iota16 = lax.iota(jnp.int32, 16)

        def chunk_step(c, cnt):
            pltpu.sync_copy(keys_hbm.at[pl.ds(row_base + c * _CHUNK, _CHUNK)], buf)

            def vec_step(i, cnt):
                kv = buf[pl.ds(i * 16, 16)]
                mask = kv >= tvec
                pos = cnt + plsc.cumsum(mask.astype(jnp.int32)) - 1
                mask = jnp.logical_and(mask, pos < _CAND)
                pid = iota16 + jnp.full((16,), pid_base + c * _CHUNK + i * 16,
                                        jnp.int32)
                plsc.store_scatter(okeys, [pos], kv, mask=mask)
                plsc.store_scatter(opids, [pos], pid, mask=mask)
                return cnt + plsc.all_reduce_population_count(mask)

            return lax.fori_loop(0, _CHUNK // 16, vec_step, cnt)

        lax.fori_loop(0, _N2 // _CHUNK, chunk_step, jnp.zeros((16,), jnp.int32))

        out_off = (r * 2 + h) * _CAND
        pltpu.sync_copy(okeys, ck_hbm.at[pl.ds(out_off, _CAND)])
        pltpu.sync_copy(opids, cp_hbm.at[pl.ds(out_off, _CAND)])


def _compact(keys_flat, thr_flat):
    mesh = plsc.VectorSubcoreMesh(core_axis_name="c", subcore_axis_name="s")
    k = pl.kernel(
        _compact_body, mesh=mesh,
        out_type=[
            jax.ShapeDtypeStruct((8 * _CAND,), jnp.float32),
            jax.ShapeDtypeStruct((8 * _CAND,), jnp.int32),
        ],
        scratch_types=[
            pltpu.VMEM((_CHUNK,), jnp.float32),
            pltpu.VMEM((16,), jnp.float32),
            pltpu.VMEM((_CAND,), jnp.float32),
            pltpu.VMEM((_CAND,), jnp.int32),
        ],
        compiler_params=pltpu.CompilerParams(needs_layout_passes=False),
    )
    return k(keys_flat, thr_flat)


def _rank_body(ck_ref, cp_ref, ckt_ref, cpt_ref, sel_ref, fid_ref):
    k = ck_ref[0]
    p = cp_ref[0]
    rows = lax.broadcasted_iota(jnp.int32, (1, _CAND), 1).astype(jnp.float32)

    def slab(s, acc):
        ks = ckt_ref[0, pl.ds(s * 128, 128), :]
        ps = cpt_ref[0, pl.ds(s * 128, 128), :]
        g = jnp.logical_or(k > ks, jnp.logical_and(k == ks, p < ps))
        rank_s = jnp.sum(g.astype(jnp.float32), axis=1, keepdims=True)
        oh = (rank_s == rows).astype(jnp.float32)
        return acc + lax.dot_general(ps.astype(jnp.float32), oh,
                                     (((0,), (0,)), ((), ())),
                                     precision=lax.Precision.HIGHEST,
                                     preferred_element_type=jnp.float32)

    acc = lax.fori_loop(0, 16, slab, jnp.zeros((1, _CAND), jnp.float32))
    pid = acc.astype(jnp.int32)
    cell = pid >> 4
    anchor = cell * _NCA + (pid & 15)
    r = pl.program_id(0)
    sel_ref[0] = r * _NUM_ANCHORS + anchor
    fid_ref[0] = r * _S + cell


def _rank(ck, cp, ckt, cpt):
    return pl.pallas_call(
        _rank_body,
        grid=(_B,),
        in_specs=[
            pl.BlockSpec((1, 1, 2 * _CAND), lambda b: (b, 0, 0)),
            pl.BlockSpec((1, 1, 2 * _CAND), lambda b: (b, 0, 0)),
            pl.BlockSpec((1, 2 * _CAND, 1), lambda b: (b, 0, 0)),
            pl.BlockSpec((1, 2 * _CAND, 1), lambda b: (b, 0, 0)),
        ],
        out_specs=[
            pl.BlockSpec((1, 1, _CAND), lambda b: (b, 0, 0)),
            pl.BlockSpec((1, 1, _CAND), lambda b: (b, 0, 0)),
        ],
        out_shape=[
            jax.ShapeDtypeStruct((_B, 1, _CAND), jnp.int32),
            jax.ShapeDtypeStruct((_B, 1, _CAND), jnp.int32),
        ],
    )(ck, cp, ckt, cpt)


def _gather_body(table_hbm, idx_hbm, out_hbm, idx_v, rows_v, sem):
    w = lax.axis_index("s") * 2 + lax.axis_index("c")
    base = w * 128
    pltpu.sync_copy(idx_hbm.at[pl.ds(base, 128)], idx_v)
    pltpu.async_copy(table_hbm.at[idx_v], rows_v, sem).wait()
    pltpu.sync_copy(rows_v, out_hbm.at[pl.ds(base, 128)])


def _gather(table, idx):
    mesh = plsc.VectorSubcoreMesh(core_axis_name="c", subcore_axis_name="s")
    k = pl.kernel(
        _gather_body, mesh=mesh,
        out_type=jax.ShapeDtypeStruct((_B * _CAND, _C), jnp.float32),
        scratch_types=[
            pltpu.VMEM((128,), jnp.int32),
            pltpu.VMEM((128, _C), jnp.float32),
            pltpu.SemaphoreType.DMA,
        ],
        compiler_params=pltpu.CompilerParams(needs_layout_passes=False),
    )
    return k(table, idx)


def kernel(feat_map0, feat_map1, feat_map2, feat_map3, W_pre, b_pre, W_proj, b_proj):
    fms = [feat_map0, feat_map1, feat_map2, feat_map3]
    B, C = _B, _C
    wprojp = jnp.concatenate(
        [W_proj, jnp.zeros((_NCA_PAD - _NCA, C), jnp.float32)], axis=0)
    bprojp = jnp.concatenate(
        [b_proj, jnp.full((_NCA_PAD - _NCA,), -jnp.inf, jnp.float32)], axis=0)

    lgs, fts = [], []
    for fm, sblk in zip(fms, (2048, 2048, 1024, 256)):
        b, c, h, w = fm.shape
        lg, ft = _proj_map(fm.reshape(b, c, h * w), W_pre, b_pre, wprojp,
                           bprojp, sblk)
        lgs.append(lg)
        fts.append(ft)

    lg16 = jnp.concatenate(lgs, axis=1)
    keys = jax.nn.sigmoid(lg16).reshape(B, _N)

    thr = _thresh(keys)
    ck, cp = _compact(keys.reshape(-1), thr.reshape(-1))
    sel, fid = _rank(ck.reshape(B, 1, 2 * _CAND), cp.reshape(B, 1, 2 * _CAND),
                     ck.reshape(B, 2 * _CAND, 1), cp.reshape(B, 2 * _CAND, 1))

    feats = jnp.concatenate(fts, axis=1).reshape(-1, C)
    gout = _gather(feats, fid.reshape(-1))

    sel_logits = lg16[:, :, :_NCA].reshape(B, -1)
    sel_ids = sel[:, 0, :_REL_THR].reshape(-1)
    sel_feats = gout.reshape(B, _CAND, C)[:, :_REL_THR].reshape(-1, C)
    return sel_logits, sel_ids, sel_feats


# speedup vs baseline: 1.2430x; 1.2430x over previous
import functools

import jax
import jax.numpy as jnp
from jax import lax
from jax.experimental import pallas as pl
from jax.experimental.pallas import tpu as pltpu
from jax.experimental.pallas import tpu_sc as plsc

_REL_THR = 1000
_NCA = 9
_NCA_PAD = 16
_B = 4
_C = 128
_S = 21760
_N = _S * _NCA
_N2 = _N // 2
_CHUNK = 19584
_CAND = 1024
_NUM_ANCHORS = _S * _NCA


def _proj_body(x_ref, wpre_ref, bpre_ref, wproj_ref, bproj_ref, lg_ref, ft_ref):
    x = x_ref[0]
    pre = lax.dot_general(x, wpre_ref[...], (((0,), (1,)), ((), ())),
                          preferred_element_type=jnp.float32)
    pre = jnp.maximum(pre + bpre_ref[...][None, :], 0.0)
    lg = lax.dot_general(pre, wproj_ref[...], (((1,), (1,)), ((), ())),
                         preferred_element_type=jnp.float32)
    lg_ref[0] = lg + bproj_ref[...][None, :]
    ft_ref[0] = x.T


def _proj_map(x, wpre, bpre, wprojp, bprojp, sblk):
    B, C, S = x.shape
    nblk = S // sblk
    grid = (B, nblk)
    return pl.pallas_call(
        _proj_body,
        grid=grid,
        in_specs=[
            pl.BlockSpec((1, C, sblk), lambda b, i: (b, 0, i)),
            pl.BlockSpec((C, C), lambda b, i: (0, 0)),
            pl.BlockSpec((C,), lambda b, i: (0,)),
            pl.BlockSpec((_NCA_PAD, C), lambda b, i: (0, 0)),
            pl.BlockSpec((_NCA_PAD,), lambda b, i: (0,)),
        ],
        out_specs=[
            pl.BlockSpec((1, sblk, _NCA_PAD), lambda b, i: (b, i, 0)),
            pl.BlockSpec((1, sblk, C), lambda b, i: (b, i, 0)),
        ],
        out_shape=[
            jax.ShapeDtypeStruct((B, S, _NCA_PAD), jnp.float32),
            jax.ShapeDtypeStruct((B, S, C), jnp.float32),
        ],
    )(x, wpre, bpre, wprojp, bprojp)


def _thresh_body(keys_ref, thr_ref):
    keys = keys_ref[...]

    def step(_, carry):
        lo, hi = carry
        mid = (lo + hi) >> 1
        midf = lax.bitcast_convert_type(mid, jnp.float32)
        cnt = jnp.sum((keys >= midf).astype(jnp.float32), axis=1, keepdims=True)
        ge_k = cnt >= float(_REL_THR)
        return jnp.where(ge_k, mid, lo), jnp.where(ge_k, hi, mid)

    lo0 = jnp.zeros((_B, 1), jnp.int32)
    hi0 = jnp.full((_B, 1), 0x3F800001, jnp.int32)
    lo, _ = lax.fori_loop(0, 31, step, (lo0, hi0))
    thr_ref[...] = jnp.broadcast_to(
        lax.bitcast_convert_type(lo, jnp.float32), (_B, 16))


def _thresh(keys):
    return pl.pallas_call(
        _thresh_body,
        in_specs=[pl.BlockSpec((_B, _N), lambda: (0, 0))],
        out_specs=pl.BlockSpec((_B, 16), lambda: (0, 0)),
        out_shape=jax.ShapeDtypeStruct((_B, 16), jnp.float32),
    )(keys)


def _compact_body(keys_hbm, thr_hbm, ck_hbm, cp_hbm, buf, tv, okeys, opids):
    w = lax.axis_index("s") * 2 + lax.axis_index("c")

    @pl.when(w < 8)
    def _():
        r = w >> 1
        h = w & 1
        pltpu.sync_copy(thr_hbm.at[pl.ds(r * 16, 16)], tv)
        tvec = tv[...]

        zeros16f = jnp.zeros((16,), jnp.float32)
        zeros16i = jnp.zeros((16,), jnp.int32)

        def zinit(i, _):
            okeys[pl.ds(i * 16, 16)] = zeros16f
            opids[pl.ds(i * 16, 16)] = zeros16i
            return 0

        lax.fori_loop(0, _CAND // 16, zinit, 0)

        row_base = r * _N + h * _N2
        pid_base = h * _N2
        iota16 = lax.iota(jnp.int32, 16)

        def chunk_step(c, cnt):
            pltpu.sync_copy(keys_hbm.at[pl.ds(row_base + c * _CHUNK, _CHUNK)], buf)

            def vec_step(i, cnt):
                kv = buf[pl.ds(i * 16, 16)]
                mask = kv >= tvec
                pos = cnt + plsc.cumsum(mask.astype(jnp.int32)) - 1
                mask = jnp.logical_and(mask, pos < _CAND)
                pid = iota16 + jnp.full((16,), pid_base + c * _CHUNK + i * 16,
                                        jnp.int32)
                plsc.store_scatter(okeys, [pos], kv, mask=mask)
                plsc.store_scatter(opids, [pos], pid, mask=mask)
                return cnt + plsc.all_reduce_population_count(mask)

            return lax.fori_loop(0, _CHUNK // 16, vec_step, cnt)

        lax.fori_loop(0, _N2 // _CHUNK, chunk_step, jnp.zeros((16,), jnp.int32))

        out_off = (r * 2 + h) * _CAND
        pltpu.sync_copy(okeys, ck_hbm.at[pl.ds(out_off, _CAND)])
        pltpu.sync_copy(opids, cp_hbm.at[pl.ds(out_off, _CAND)])


def _compact(keys_flat, thr_flat):
    mesh = plsc.VectorSubcoreMesh(core_axis_name="c", subcore_axis_name="s")
    k = pl.kernel(
        _compact_body, mesh=mesh,
        out_type=[
            jax.ShapeDtypeStruct((8 * _CAND,), jnp.float32),
            jax.ShapeDtypeStruct((8 * _CAND,), jnp.int32),
        ],
        scratch_types=[
            pltpu.VMEM((_CHUNK,), jnp.float32),
            pltpu.VMEM((16,), jnp.float32),
            pltpu.VMEM((_CAND,), jnp.float32),
            pltpu.VMEM((_CAND,), jnp.int32),
        ],
        compiler_params=pltpu.CompilerParams(needs_layout_passes=False),
    )
    return k(keys_flat, thr_flat)


def _rank_body(ck_ref, cp_ref, ckt_ref, cpt_ref, sel_ref, fid_ref):
    k = ck_ref[0]
    p = cp_ref[0]
    rows = lax.broadcasted_iota(jnp.int32, (1, _CAND), 1).astype(jnp.float32)

    def slab(s, acc):
        ks = ckt_ref[0, pl.ds(s * 128, 128), :]
        ps = cpt_ref[0, pl.ds(s * 128, 128), :]
        g = jnp.logical_or(k > ks, jnp.logical_and(k == ks, p < ps))
        rank_s = jnp.sum(g.astype(jnp.float32), axis=1, keepdims=True)
        oh = (rank_s == rows).astype(jnp.float32)
        return acc + lax.dot_general(ps.astype(jnp.float32), oh,
                                     (((0,), (0,)), ((), ())),
                                     precision=lax.Precision.HIGHEST,
                                     preferred_element_type=jnp.float32)

    acc = lax.fori_loop(0, 16, slab, jnp.zeros((1, _CAND), jnp.float32))
    pid = acc.astype(jnp.int32)
    cell = jnp.floor((acc + 0.5) * (1.0 / 9.0)).astype(jnp.int32)
    r = pl.program_id(0)
    sel_ref[0] = r * _NUM_ANCHORS + pid
    fid_ref[0] = r * _S + cell


def _rank(ck, cp, ckt, cpt):
    return pl.pallas_call(
        _rank_body,
        grid=(_B,),
        in_specs=[
            pl.BlockSpec((1, 1, 2 * _CAND), lambda b: (b, 0, 0)),
            pl.BlockSpec((1, 1, 2 * _CAND), lambda b: (b, 0, 0)),
            pl.BlockSpec((1, 2 * _CAND, 1), lambda b: (b, 0, 0)),
            pl.BlockSpec((1, 2 * _CAND, 1), lambda b: (b, 0, 0)),
        ],
        out_specs=[
            pl.BlockSpec((1, 1, _CAND), lambda b: (b, 0, 0)),
            pl.BlockSpec((1, 1, _CAND), lambda b: (b, 0, 0)),
        ],
        out_shape=[
            jax.ShapeDtypeStruct((_B, 1, _CAND), jnp.int32),
            jax.ShapeDtypeStruct((_B, 1, _CAND), jnp.int32),
        ],
    )(ck, cp, ckt, cpt)


def _gather_body(table_hbm, idx_hbm, out_hbm, idx_v, rows_v, sem):
    w = lax.axis_index("s") * 2 + lax.axis_index("c")
    base = w * 128
    pltpu.sync_copy(idx_hbm.at[pl.ds(base, 128)], idx_v)
    pltpu.async_copy(table_hbm.at[idx_v], rows_v, sem).wait()
    pltpu.sync_copy(rows_v, out_hbm.at[pl.ds(base, 128)])


def _gather(table, idx):
    mesh = plsc.VectorSubcoreMesh(core_axis_name="c", subcore_axis_name="s")
    k = pl.kernel(
        _gather_body, mesh=mesh,
        out_type=jax.ShapeDtypeStruct((_B * _CAND, _C), jnp.float32),
        scratch_types=[
            pltpu.VMEM((128,), jnp.int32),
            pltpu.VMEM((128, _C), jnp.float32),
            pltpu.SemaphoreType.DMA,
        ],
        compiler_params=pltpu.CompilerParams(needs_layout_passes=False),
    )
    return k(table, idx)


def kernel(feat_map0, feat_map1, feat_map2, feat_map3, W_pre, b_pre, W_proj, b_proj):
    fms = [feat_map0, feat_map1, feat_map2, feat_map3]
    B, C = _B, _C
    wprojp = jnp.concatenate(
        [W_proj, jnp.zeros((_NCA_PAD - _NCA, C), jnp.float32)], axis=0)
    bprojp = jnp.concatenate(
        [b_proj, jnp.full((_NCA_PAD - _NCA,), -jnp.inf, jnp.float32)], axis=0)

    lgs, fts = [], []
    for fm, sblk in zip(fms, (2048, 2048, 1024, 256)):
        b, c, h, w = fm.shape
        lg, ft = _proj_map(fm.reshape(b, c, h * w), W_pre, b_pre, wprojp,
                           bprojp, sblk)
        lgs.append(lg)
        fts.append(ft)

    lg16 = jnp.concatenate(lgs, axis=1)
    sel_logits = lg16[:, :, :_NCA].reshape(B, _N)
    keys = jax.nn.sigmoid(sel_logits)

    thr = _thresh(keys)
    ck, cp = _compact(keys.reshape(-1), thr.reshape(-1))
    sel, fid = _rank(ck.reshape(B, 1, 2 * _CAND), cp.reshape(B, 1, 2 * _CAND),
                     ck.reshape(B, 2 * _CAND, 1), cp.reshape(B, 2 * _CAND, 1))

    feats = jnp.concatenate(fts, axis=1).reshape(-1, C)
    gout = _gather(feats, fid.reshape(-1))

    sel_ids = sel[:, 0, :_REL_THR].reshape(-1)
    sel_feats = gout.reshape(B, _CAND, C)[:, :_REL_THR].reshape(-1, C)
    return sel_logits, sel_ids, sel_feats
